# baseline (speedup 1.0000x reference)
import functools

import jax
import jax.numpy as jnp
from jax.experimental import pallas as pl
from jax.experimental.pallas import tpu as pltpu

_G = 2
_K = 320
_D = 256
_DG = _D // _G
_NB = 1024

_IDX_MASK = 0x1FF
_INT_MAX = 0x7FFFFFFF


def _split2(c):
    hi = c.astype(jnp.bfloat16)
    lo = (c - hi.astype(jnp.float32)).astype(jnp.bfloat16)
    return hi, lo


def _split3(c):
    hi = c.astype(jnp.bfloat16)
    r1 = c - hi.astype(jnp.float32)
    mid = r1.astype(jnp.bfloat16)
    lo = (r1 - mid.astype(jnp.float32)).astype(jnp.bfloat16)
    return hi, mid, lo


def _mm(a, b, dims):
    return jax.lax.dot_general(a, b, (dims, ((), ())),
                               preferred_element_type=jnp.float32)


def _vq_body(x_ref, ct_ref, dist_ref, enc_ref, quant_ref, ppl_ref,
             counts_ref):
    i = pl.program_id(0)
    nsteps = pl.num_programs(0)

    @pl.when(i == 0)
    def _init():
        counts_ref[...] = jnp.zeros_like(counts_ref)

    x = x_ref[...]
    iota0 = jax.lax.broadcasted_iota(jnp.int32, (_K, _NB), 0)
    ones_col = jnp.ones((_NB, 1), jnp.float32)
    for g in range(_G):
        xg = x[:, g * _DG:(g + 1) * _DG]
        ctg = ct_ref[g]
        c2 = jnp.sum(ctg * ctg, axis=0, keepdims=True)
        x2 = jnp.sum(xg * xg, axis=1, keepdims=True)

        a_aug = jnp.concatenate(
            [jnp.transpose(-2.0 * ctg), jnp.transpose(c2),
             jnp.ones((_K, 1), jnp.float32)], axis=1)
        x_aug = jnp.concatenate([xg, ones_col, x2], axis=1)
        ah, al = _split2(a_aug)
        xh, xl = _split2(x_aug)
        tdims = ((1,), (1,))
        dist = _mm(ah, xh, tdims) + (_mm(ah, xl, tdims) + _mm(al, xh, tdims))
        dist_ref[g, 0] = dist

        bits = jax.lax.bitcast_convert_type(dist, jnp.int32)
        key = jax.lax.bitwise_or(
            jax.lax.bitwise_and(bits, jnp.int32(~_IDX_MASK)), iota0)
        k1 = jnp.min(key, axis=0, keepdims=True)
        i1 = jax.lax.bitwise_and(k1, jnp.int32(_IDX_MASK))
        keym = jnp.where(key == k1, jnp.int32(_INT_MAX), key)
        k2 = jnp.min(keym, axis=0, keepdims=True)
        i2 = jax.lax.bitwise_and(k2, jnp.int32(_IDX_MASK))

        e1 = (iota0 == i1).astype(jnp.bfloat16)
        e2 = (iota0 == i2).astype(jnp.bfloat16)
        hi, mid, lo = _split3(ctg)
        sdims = ((1,), (0,))

        def _sel(e):
            return (_mm(hi, e, sdims)
                    + (_mm(mid, e, sdims) + _mm(lo, e, sdims)))

        c1 = _sel(e1)
        c2v = _sel(e2)
        xt = jnp.transpose(xg)
        d1 = jnp.sum((xt - c1) ** 2, axis=0, keepdims=True)
        d2 = jnp.sum((xt - c2v) ** 2, axis=0, keepdims=True)
        take1 = (d1 < d2) | ((d1 == d2) & (i1 < i2))

        win = jnp.where(take1, i1, i2)
        enc = (iota0 == win).astype(jnp.float32)
        enc_ref[g, 0] = enc
        quant_ref[:, g * _DG:(g + 1) * _DG] = jnp.transpose(
            jnp.where(take1, c1, c2v))
        counts_ref[g] += jnp.sum(enc, axis=1, keepdims=True)

    @pl.when(i == nsteps - 1)
    def _fin():
        avg = counts_ref[...] / jnp.float32(nsteps * _NB)
        ent = -jnp.sum(avg * jnp.log(avg + 1e-10), axis=1)
        ppl_ref[...] = jnp.mean(jnp.exp(ent)).reshape(1, 1)


@jax.jit
def kernel(hidden_states, codevectors):
    B, T, D = hidden_states.shape
    N = B * T
    TB = T // _NB
    x = hidden_states.reshape(N, D)
    ct = jnp.transpose(codevectors, (0, 2, 1))
    grid = (N // _NB,)
    dist, enc, quant, ppl = pl.pallas_call(
        _vq_body,
        grid=grid,
        in_specs=[
            pl.BlockSpec((_NB, _D), lambda i: (i, 0)),
            pl.BlockSpec((_G, _DG, _K), lambda i: (0, 0, 0)),
        ],
        out_specs=[
            pl.BlockSpec((_G, 1, _K, _NB), lambda i: (0, i // TB, 0, i % TB)),
            pl.BlockSpec((_G, 1, _K, _NB), lambda i: (0, i // TB, 0, i % TB)),
            pl.BlockSpec((_NB, _D), lambda i: (i, 0)),
            pl.BlockSpec((1, 1), lambda i: (0, 0)),
        ],
        out_shape=[
            jax.ShapeDtypeStruct((_G, B, _K, T), jnp.float32),
            jax.ShapeDtypeStruct((_G, B, _K, T), jnp.float32),
            jax.ShapeDtypeStruct((N, _D), jnp.float32),
            jax.ShapeDtypeStruct((1, 1), jnp.float32),
        ],
        scratch_shapes=[pltpu.VMEM((_G, _K, 1), jnp.float32)],
    )(x, ct)
    quantized = quant.reshape(B, T, D)
    encodings = jnp.transpose(enc, (0, 1, 3, 2))
    distances = jnp.transpose(dist, (0, 1, 3, 2))
    perplexity = ppl.reshape(())
    return quantized, encodings, distances, perplexity


# speedup vs baseline: 1.0312x; 1.0312x over previous
import functools

import jax
import jax.numpy as jnp
from jax.experimental import pallas as pl
from jax.experimental.pallas import tpu as pltpu

_G = 2
_K = 320
_D = 256
_DG = _D // _G
_NB = 2048

_IDX_MASK = 0x1FF
_INT_MAX = 0x7FFFFFFF


def _split2(c):
    hi = c.astype(jnp.bfloat16)
    lo = (c - hi.astype(jnp.float32)).astype(jnp.bfloat16)
    return hi, lo


def _split3(c):
    hi = c.astype(jnp.bfloat16)
    r1 = c - hi.astype(jnp.float32)
    mid = r1.astype(jnp.bfloat16)
    lo = (r1 - mid.astype(jnp.float32)).astype(jnp.bfloat16)
    return hi, mid, lo


def _mm(a, b, dims):
    return jax.lax.dot_general(a, b, (dims, ((), ())),
                               preferred_element_type=jnp.float32)


def _vq_body(x_ref, ct_ref, dist_ref, enc_ref, quant_ref, ppl_ref,
             counts_ref):
    i = pl.program_id(0)
    g = pl.program_id(1)
    nsteps = pl.num_programs(0)

    @pl.when((i == 0) & (g == 0))
    def _init():
        counts_ref[...] = jnp.zeros_like(counts_ref)

    xg = x_ref[...]
    ctg = ct_ref[0]
    iota0 = jax.lax.broadcasted_iota(jnp.int32, (_K, _NB), 0)
    c2 = jnp.sum(ctg * ctg, axis=0, keepdims=True)
    x2 = jnp.sum(xg * xg, axis=1, keepdims=True)

    a_aug = jnp.concatenate(
        [jnp.transpose(-2.0 * ctg), jnp.transpose(c2),
         jnp.ones((_K, 1), jnp.float32)], axis=1)
    x_aug = jnp.concatenate(
        [xg, jnp.ones((_NB, 1), jnp.float32), x2], axis=1)
    ah, al = _split2(a_aug)
    xh, xl = _split2(x_aug)
    tdims = ((1,), (1,))
    dist = _mm(ah, xh, tdims) + (_mm(ah, xl, tdims) + _mm(al, xh, tdims))
    dist_ref[0, 0] = dist

    bits = jax.lax.bitcast_convert_type(dist, jnp.int32)
    key = jax.lax.bitwise_or(
        jax.lax.bitwise_and(bits, jnp.int32(~_IDX_MASK)), iota0)
    k1 = jnp.min(key, axis=0, keepdims=True)
    i1 = jax.lax.bitwise_and(k1, jnp.int32(_IDX_MASK))
    keym = jnp.where(key == k1, jnp.int32(_INT_MAX), key)
    k2 = jnp.min(keym, axis=0, keepdims=True)
    i2 = jax.lax.bitwise_and(k2, jnp.int32(_IDX_MASK))

    e1 = (iota0 == i1).astype(jnp.bfloat16)
    e2 = (iota0 == i2).astype(jnp.bfloat16)
    hi, mid, lo = _split3(ctg)
    sdims = ((1,), (0,))

    def _sel(e):
        return _mm(hi, e, sdims) + (_mm(mid, e, sdims) + _mm(lo, e, sdims))

    c1 = _sel(e1)
    c2v = _sel(e2)
    xt = jnp.transpose(xg)
    d1 = jnp.sum((xt - c1) ** 2, axis=0, keepdims=True)
    d2 = jnp.sum((xt - c2v) ** 2, axis=0, keepdims=True)
    take1 = (d1 < d2) | ((d1 == d2) & (i1 < i2))

    win = jnp.where(take1, i1, i2)
    enc = (iota0 == win).astype(jnp.float32)
    enc_ref[0, 0] = enc
    quant_ref[...] = jnp.transpose(jnp.where(take1, c1, c2v))
    counts_ref[pl.ds(g, 1)] += jnp.sum(enc, axis=1, keepdims=True)

    @pl.when((i == nsteps - 1) & (g == _G - 1))
    def _fin():
        avg = counts_ref[...] / jnp.float32(nsteps * _NB)
        ent = -jnp.sum(avg * jnp.log(avg + 1e-10), axis=1)
        ppl_ref[...] = jnp.mean(jnp.exp(ent)).reshape(1, 1)


@jax.jit
def kernel(hidden_states, codevectors):
    B, T, D = hidden_states.shape
    N = B * T
    TB = T // _NB
    x = hidden_states.reshape(N, D)
    ct = jnp.transpose(codevectors, (0, 2, 1))
    grid = (N // _NB, _G)
    dist, enc, quant, ppl = pl.pallas_call(
        _vq_body,
        grid=grid,
        in_specs=[
            pl.BlockSpec((_NB, _DG), lambda i, g: (i, g)),
            pl.BlockSpec((1, _DG, _K), lambda i, g: (g, 0, 0)),
        ],
        out_specs=[
            pl.BlockSpec((1, 1, _K, _NB),
                         lambda i, g: (g, i // TB, 0, i % TB)),
            pl.BlockSpec((1, 1, _K, _NB),
                         lambda i, g: (g, i // TB, 0, i % TB)),
            pl.BlockSpec((_NB, _DG), lambda i, g: (i, g)),
            pl.BlockSpec((1, 1), lambda i, g: (0, 0)),
        ],
        out_shape=[
            jax.ShapeDtypeStruct((_G, B, _K, T), jnp.float32),
            jax.ShapeDtypeStruct((_G, B, _K, T), jnp.float32),
            jax.ShapeDtypeStruct((N, _D), jnp.float32),
            jax.ShapeDtypeStruct((1, 1), jnp.float32),
        ],
        scratch_shapes=[pltpu.VMEM((_G, _K, 1), jnp.float32)],
    )(x, ct)
    quantized = quant.reshape(B, T, D)
    encodings = jnp.transpose(enc, (0, 1, 3, 2))
    distances = jnp.transpose(dist, (0, 1, 3, 2))
    perplexity = ppl.reshape(())
    return quantized, encodings, distances, perplexity


# speedup vs baseline: 1.0611x; 1.0291x over previous
import functools

import jax
import jax.numpy as jnp
from jax.experimental import pallas as pl
from jax.experimental.pallas import tpu as pltpu

_G = 2
_K = 320
_D = 256
_DG = _D // _G
_NB = 2048

_IDX_MASK = 0x1FF
_INT_MAX = 0x7FFFFFFF


def _split2(c):
    hi = c.astype(jnp.bfloat16)
    lo = (c - hi.astype(jnp.float32)).astype(jnp.bfloat16)
    return hi, lo


def _split3(c):
    hi = c.astype(jnp.bfloat16)
    r1 = c - hi.astype(jnp.float32)
    mid = r1.astype(jnp.bfloat16)
    lo = (r1 - mid.astype(jnp.float32)).astype(jnp.bfloat16)
    return hi, mid, lo


def _mm(a, b, dims):
    return jax.lax.dot_general(a, b, (dims, ((), ())),
                               preferred_element_type=jnp.float32)


def _vq_body(x_ref, ct_ref, dist_ref, enc_ref, quant_ref, ppl_ref,
             counts_ref):
    i = pl.program_id(0)
    nsteps = pl.num_programs(0)

    @pl.when(i == 0)
    def _init():
        counts_ref[...] = jnp.zeros_like(counts_ref)

    x = x_ref[...]
    iota0 = jax.lax.broadcasted_iota(jnp.int32, (_K, _NB), 0)
    ones_col = jnp.ones((_NB, 1), jnp.float32)
    for g in range(_G):
        xg = x[:, g * _DG:(g + 1) * _DG]
        ctg = ct_ref[g]
        c2 = jnp.sum(ctg * ctg, axis=0, keepdims=True)
        x2 = jnp.sum(xg * xg, axis=1, keepdims=True)

        a_aug = jnp.concatenate(
            [jnp.transpose(-2.0 * ctg), jnp.transpose(c2),
             jnp.ones((_K, 1), jnp.float32)], axis=1)
        x_aug = jnp.concatenate([xg, ones_col, x2], axis=1)
        ah, al = _split2(a_aug)
        xh, xl = _split2(x_aug)
        tdims = ((1,), (1,))
        dist = _mm(ah, xh, tdims) + (_mm(ah, xl, tdims) + _mm(al, xh, tdims))
        dist_ref[g, 0] = dist

        bits = jax.lax.bitcast_convert_type(dist, jnp.int32)
        key = jax.lax.bitwise_or(
            jax.lax.bitwise_and(bits, jnp.int32(~_IDX_MASK)), iota0)
        k1 = jnp.min(key, axis=0, keepdims=True)
        i1 = jax.lax.bitwise_and(k1, jnp.int32(_IDX_MASK))
        keym = jnp.where(key == k1, jnp.int32(_INT_MAX), key)
        k2 = jnp.min(keym, axis=0, keepdims=True)
        i2 = jax.lax.bitwise_and(k2, jnp.int32(_IDX_MASK))

        e1 = (iota0 == i1).astype(jnp.bfloat16)
        e2 = (iota0 == i2).astype(jnp.bfloat16)
        hi, mid, lo = _split3(ctg)
        sdims = ((1,), (0,))

        def _sel(e):
            return (_mm(hi, e, sdims)
                    + (_mm(mid, e, sdims) + _mm(lo, e, sdims)))

        c1 = _sel(e1)
        c2v = _sel(e2)
        xt = jnp.transpose(xg)
        d1 = jnp.sum((xt - c1) ** 2, axis=0, keepdims=True)
        d2 = jnp.sum((xt - c2v) ** 2, axis=0, keepdims=True)
        take1 = (d1 < d2) | ((d1 == d2) & (i1 < i2))

        win = jnp.where(take1, i1, i2)
        enc = (iota0 == win).astype(jnp.float32)
        enc_ref[g, 0] = enc
        quant_ref[:, g * _DG:(g + 1) * _DG] = jnp.transpose(
            jnp.where(take1, c1, c2v))
        counts_ref[g] += jnp.sum(enc, axis=1, keepdims=True)

    @pl.when(i == nsteps - 1)
    def _fin():
        avg = counts_ref[...] / jnp.float32(nsteps * _NB)
        ent = -jnp.sum(avg * jnp.log(avg + 1e-10), axis=1)
        ppl_ref[...] = jnp.mean(jnp.exp(ent)).reshape(1, 1)


@jax.jit
def kernel(hidden_states, codevectors):
    B, T, D = hidden_states.shape
    N = B * T
    TB = T // _NB
    x = hidden_states.reshape(N, D)
    ct = jnp.transpose(codevectors, (0, 2, 1))
    grid = (N // _NB,)
    dist, enc, quant, ppl = pl.pallas_call(
        _vq_body,
        grid=grid,
        in_specs=[
            pl.BlockSpec((_NB, _D), lambda i: (i, 0)),
            pl.BlockSpec((_G, _DG, _K), lambda i: (0, 0, 0)),
        ],
        out_specs=[
            pl.BlockSpec((_G, 1, _K, _NB), lambda i: (0, i // TB, 0, i % TB)),
            pl.BlockSpec((_G, 1, _K, _NB), lambda i: (0, i // TB, 0, i % TB)),
            pl.BlockSpec((_NB, _D), lambda i: (i, 0)),
            pl.BlockSpec((1, 1), lambda i: (0, 0)),
        ],
        out_shape=[
            jax.ShapeDtypeStruct((_G, B, _K, T), jnp.float32),
            jax.ShapeDtypeStruct((_G, B, _K, T), jnp.float32),
            jax.ShapeDtypeStruct((N, _D), jnp.float32),
            jax.ShapeDtypeStruct((1, 1), jnp.float32),
        ],
        scratch_shapes=[pltpu.VMEM((_G, _K, 1), jnp.float32)],
    )(x, ct)
    quantized = quant.reshape(B, T, D)
    encodings = jnp.transpose(enc, (0, 1, 3, 2))
    distances = jnp.transpose(dist, (0, 1, 3, 2))
    perplexity = ppl.reshape(())
    return quantized, encodings, distances, perplexity
